# baseline (speedup 1.0000x reference)
import functools

import jax
import jax.numpy as jnp
from jax import lax
from jax.experimental import pallas as pl
from jax.experimental.pallas import tpu as pltpu
from jax.experimental.pallas import tpu_sc as plsc

_SC_CORES = 2
_SC_SUBCORES = 16


def _ce_kernel(logits_ref, tgt_ref, loss_ref):
    x = logits_ref[...]
    t = tgt_ref[...]
    m = jnp.max(x, axis=0, keepdims=True)
    e = jnp.exp(x - m)
    s = jnp.sum(e, axis=0, keepdims=True)
    lse = m[0] + jnp.log(s[0])
    cls = jax.lax.broadcasted_iota(jnp.int32, x.shape, 0)
    tl = jnp.sum(jnp.where(cls == t[None], x, 0.0), axis=0)
    loss_ref[...] = lse - tl


def _sc_select_body(loss_hbm, tgt_hbm, out_hbm, loss_v, tgt_v, hist_c, hist_s, out_v):
    n = loss_v.shape[0]
    nchunk = n // 16
    wid = lax.axis_index("s") * _SC_CORES + lax.axis_index("c")
    pltpu.sync_copy(loss_hbm.at[wid], loss_v)
    pltpu.sync_copy(tgt_hbm.at[wid], tgt_v)
    lane = lax.iota(jnp.int32, 16)
    zc = jnp.zeros((16,), jnp.int32)
    zs = jnp.zeros((16,), jnp.float32)
    one = jnp.ones((16,), jnp.int32)

    def zero_body(d, _):
        hist_c[pl.ds(d * 16, 16)] = zc
        hist_s[pl.ds(d * 16, 16)] = zs
        return 0

    lax.fori_loop(0, 512, zero_body, 0, unroll=8)

    def pass_a(i, carry):
        cnt_acc, sum_acc = carry
        lv = loss_v[pl.ds(i * 16, 16)]
        tv = tgt_v[pl.ds(i * 16, 16)]
        m = tv > 0
        cnt_acc = cnt_acc + jnp.where(m, jnp.int32(1), jnp.int32(0))
        sum_acc = sum_acc + jnp.where(m, lv, jnp.float32(0.0))
        con = jnp.where(m, jnp.float32(0.0), jnp.maximum(lv, jnp.float32(0.0)))
        loss_v[pl.ds(i * 16, 16)] = con
        bits = plsc.bitcast(con, jnp.int32)
        idx = ((bits >> 22) << 4) + lane
        plsc.addupdate_scatter(hist_c, [idx], one)
        plsc.addupdate_scatter(hist_s, [idx], con)
        return cnt_acc, sum_acc

    cnt_acc, sum_acc = lax.fori_loop(
        0, nchunk, pass_a,
        (jnp.zeros((16,), jnp.int32), jnp.zeros((16,), jnp.float32)),
        unroll=8)
    pos_num = jnp.sum(cnt_acc)
    pos_sum = jnp.sum(sum_acc)
    k = jnp.minimum(3 * pos_num, n)

    def fill_level(shift, prefix, mask_shift):
        def body(i, _):
            cv = loss_v[pl.ds(i * 16, 16)]
            bits = plsc.bitcast(cv, jnp.int32)
            m = (bits >> mask_shift) == prefix
            idx = (((bits >> shift) & 0x1FF) << 4) + lane
            plsc.addupdate_scatter(hist_c, [idx], one, mask=m)
            plsc.addupdate_scatter(hist_s, [idx], cv, mask=m)
            return 0

        lax.fori_loop(0, nchunk, body, 0, unroll=8)

    def scan_level(nbuckets, k_rem, s_above):
        nch = nbuckets // 16

        def chunk_body(j, carry):
            found, cstar, krem_c, s_ab, cum = carry
            c = nch - 1 - j
            base = c * 256

            def acc_body(l, a):
                return (a[0] + hist_c[pl.ds(base + l * 16, 16)],
                        a[1] + hist_s[pl.ds(base + l * 16, 16)])

            cc, ss = lax.fori_loop(0, 16, acc_body, (zc, zs), unroll=16)
            tot_c = jnp.sum(cc)
            tot_s = jnp.sum(ss)
            newcum = cum + tot_c
            cross = jnp.logical_and(jnp.logical_not(found), newcum >= k_rem)
            cstar = jnp.where(cross, c, cstar)
            krem_c = jnp.where(cross, k_rem - cum, krem_c)
            take_all = jnp.logical_and(jnp.logical_not(found), newcum < k_rem)
            s_ab = jnp.where(take_all, s_ab + tot_s, s_ab)
            found = jnp.logical_or(found, cross)
            return found, cstar, krem_c, s_ab, newcum

        init = (jnp.bool_(False), jnp.int32(0), jnp.int32(0), s_above,
                jnp.int32(0))
        _, cstar, krem_c, s_above, _ = lax.fori_loop(0, nch, chunk_body, init)

        def bucket_body(j, carry):
            found, dstar, krem_d, s_ab, cum = carry
            d = cstar * 16 + (15 - j)
            c_d = jnp.sum(hist_c[pl.ds(d * 16, 16)])
            s_d = jnp.sum(hist_s[pl.ds(d * 16, 16)])
            newcum = cum + c_d
            cross = jnp.logical_and(jnp.logical_not(found), newcum >= krem_c)
            dstar = jnp.where(cross, d, dstar)
            krem_d = jnp.where(cross, krem_c - cum, krem_d)
            take_all = jnp.logical_and(jnp.logical_not(found), newcum < krem_c)
            s_ab = jnp.where(take_all, s_ab + s_d, s_ab)
            found = jnp.logical_or(found, cross)
            return found, dstar, krem_d, s_ab, newcum

        init = (jnp.bool_(False), jnp.int32(0), jnp.int32(0), s_above,
                jnp.int32(0))
        _, beta, krem, s_above, _ = lax.fori_loop(0, 16, bucket_body, init)
        return beta, krem, s_above

    beta1, krem, s_above = scan_level(512, k, jnp.float32(0.0))
    lax.fori_loop(0, 512, zero_body, 0, unroll=8)
    fill_level(13, beta1, 22)
    beta2, krem, s_above = scan_level(512, krem, s_above)
    p2 = (beta1 << 9) | beta2
    lax.fori_loop(0, 512, zero_body, 0, unroll=8)
    fill_level(4, p2, 13)
    beta3, krem, s_above = scan_level(512, krem, s_above)
    p3 = (p2 << 9) | beta3

    def zero_body4(d, _):
        hist_c[pl.ds(d * 16, 16)] = zc
        hist_s[pl.ds(d * 16, 16)] = zs
        return 0

    lax.fori_loop(0, 16, zero_body4, 0, unroll=8)

    def fill4(i, _):
        cv = loss_v[pl.ds(i * 16, 16)]
        bits = plsc.bitcast(cv, jnp.int32)
        m = (bits >> 4) == p3
        idx = ((bits & 0xF) << 4) + lane
        plsc.addupdate_scatter(hist_c, [idx], one, mask=m)
        plsc.addupdate_scatter(hist_s, [idx], cv, mask=m)
        return 0

    lax.fori_loop(0, nchunk, fill4, 0, unroll=8)
    beta4, krem, s_above = scan_level(16, krem, s_above)
    t_bits = (p3 << 4) | beta4

    t_vec = plsc.bitcast(jnp.full((16,), t_bits, jnp.int32), jnp.float32)
    k_vec = jnp.full((16,), krem, jnp.int32).astype(jnp.float32)
    tie = jnp.where(jnp.full((16,), k, jnp.int32) > 0,
                    t_vec * k_vec, jnp.float32(0.0))
    out_v[...] = jnp.full((16,), pos_sum + s_above, jnp.float32) + tie
    pltpu.sync_copy(out_v, out_hbm.at[wid])


def kernel(pred_loc, pred_bclass, true_loc_vec, true_bclass):
    del pred_loc, true_loc_vec
    b, c, n = pred_bclass.shape
    pb_t = jnp.transpose(pred_bclass, (1, 0, 2))

    rb = 8
    nb = 4096
    nt = pl.cdiv(n, nb)
    loss = pl.pallas_call(
        _ce_kernel,
        grid=(b // rb, nt),
        in_specs=[
            pl.BlockSpec((c, rb, nb), lambda i, j: (0, i, j)),
            pl.BlockSpec((rb, nb), lambda i, j: (i, j)),
        ],
        out_specs=pl.BlockSpec((rb, nb), lambda i, j: (i, j)),
        out_shape=jax.ShapeDtypeStruct((b, n), jnp.float32),
    )(pb_t, true_bclass)

    sc_select = pl.kernel(
        _sc_select_body,
        out_type=jax.ShapeDtypeStruct((b, 16), jnp.float32),
        mesh=plsc.VectorSubcoreMesh(
            core_axis_name="c", subcore_axis_name="s",
            num_cores=_SC_CORES, num_subcores=_SC_SUBCORES),
        scratch_types=[
            pltpu.VMEM((n,), jnp.float32),
            pltpu.VMEM((n,), jnp.int32),
            pltpu.VMEM((8192,), jnp.int32),
            pltpu.VMEM((8192,), jnp.float32),
            pltpu.VMEM((16,), jnp.float32),
        ],
        compiler_params=pltpu.CompilerParams(needs_layout_passes=False),
    )
    out = sc_select(loss, true_bclass)
    return out[:, 0]


# speedup vs baseline: 1.0253x; 1.0253x over previous
import jax
import jax.numpy as jnp
from jax import lax
from jax.experimental import pallas as pl
from jax.experimental.pallas import tpu as pltpu
from jax.experimental.pallas import tpu_sc as plsc

_SC_CORES = 2
_SC_SUBCORES = 16


def _ce_kernel(logits_ref, tgt_ref, loss_ref):
    x = logits_ref[...]
    t = tgt_ref[...]
    m = jnp.max(x, axis=0, keepdims=True)
    e = jnp.exp(x - m)
    s = jnp.sum(e, axis=0, keepdims=True)
    lse = m[0] + jnp.log(s[0])
    cls = jax.lax.broadcasted_iota(jnp.int32, x.shape, 0)
    tl = jnp.sum(jnp.where(cls == t[None], x, 0.0), axis=0)
    loss_ref[...] = lse - tl


def _sc_select_body(loss_hbm, tgt_hbm, out_hbm, loss_v, tgt_v, hist_c, out_v):
    n = loss_v.shape[0]
    nchunk = n // 16
    wid = lax.axis_index("s") * _SC_CORES + lax.axis_index("c")
    pltpu.sync_copy(loss_hbm.at[wid], loss_v)
    pltpu.sync_copy(tgt_hbm.at[wid], tgt_v)
    lane = lax.iota(jnp.int32, 16)
    zc = jnp.zeros((16,), jnp.int32)
    zs = jnp.zeros((16,), jnp.float32)
    one = jnp.ones((16,), jnp.int32)

    def zero_body(d, _):
        hist_c[pl.ds(d * 16, 16)] = zc
        return 0

    lax.fori_loop(0, 512, zero_body, 0, unroll=8)

    def pass_a(i, carry):
        cnt_acc, sum_acc = carry
        lv = loss_v[pl.ds(i * 16, 16)]
        tv = tgt_v[pl.ds(i * 16, 16)]
        m = tv > 0
        cnt_acc = cnt_acc + jnp.where(m, jnp.int32(1), jnp.int32(0))
        sum_acc = sum_acc + jnp.where(m, lv, jnp.float32(0.0))
        con = jnp.where(m, jnp.float32(0.0), jnp.maximum(lv, jnp.float32(0.0)))
        loss_v[pl.ds(i * 16, 16)] = con
        bits = plsc.bitcast(con, jnp.int32)
        idx = ((bits >> 22) << 4) + lane
        plsc.addupdate_scatter(hist_c, [idx], one)
        return cnt_acc, sum_acc

    cnt_acc, sum_acc = lax.fori_loop(
        0, nchunk, pass_a,
        (jnp.zeros((16,), jnp.int32), jnp.zeros((16,), jnp.float32)),
        unroll=8)
    pos_num = jnp.sum(cnt_acc)
    pos_sum = jnp.sum(sum_acc)
    k = jnp.minimum(3 * pos_num, n)

    def fill_level(shift, dmask, prefix, mask_shift):
        def body(i, _):
            cv = loss_v[pl.ds(i * 16, 16)]
            bits = plsc.bitcast(cv, jnp.int32)
            m = (bits >> mask_shift) == prefix
            idx = (((bits >> shift) & dmask) << 4) + lane
            plsc.addupdate_scatter(hist_c, [idx], one, mask=m)
            return 0

        lax.fori_loop(0, nchunk, body, 0, unroll=8)

    def scan_level(nbuckets, k_rem):
        nch = nbuckets // 16

        def chunk_body(j, carry):
            found, cstar, krem_c, cum = carry
            c = nch - 1 - j
            base = c * 256

            def acc_body(l, a):
                return a + hist_c[pl.ds(base + l * 16, 16)]

            cc = lax.fori_loop(0, 16, acc_body, zc, unroll=16)
            tot_c = jnp.sum(cc)
            newcum = cum + tot_c
            cross = jnp.logical_and(jnp.logical_not(found), newcum >= k_rem)
            cstar = jnp.where(cross, c, cstar)
            krem_c = jnp.where(cross, k_rem - cum, krem_c)
            found = jnp.logical_or(found, cross)
            return found, cstar, krem_c, newcum

        init = (jnp.bool_(False), jnp.int32(0), jnp.int32(0), jnp.int32(0))
        _, cstar, krem_c, _ = lax.fori_loop(0, nch, chunk_body, init)

        def bucket_body(j, carry):
            found, dstar, krem_d, cum = carry
            d = cstar * 16 + (15 - j)
            c_d = jnp.sum(hist_c[pl.ds(d * 16, 16)])
            newcum = cum + c_d
            cross = jnp.logical_and(jnp.logical_not(found), newcum >= krem_c)
            dstar = jnp.where(cross, d, dstar)
            krem_d = jnp.where(cross, krem_c - cum, krem_d)
            found = jnp.logical_or(found, cross)
            return found, dstar, krem_d, newcum

        init = (jnp.bool_(False), jnp.int32(0), jnp.int32(0), jnp.int32(0))
        _, beta, krem, _ = lax.fori_loop(0, 16, bucket_body, init)
        return beta, krem

    beta1, krem = scan_level(512, k)
    lax.fori_loop(0, 512, zero_body, 0, unroll=8)
    fill_level(13, 0x1FF, beta1, 22)
    beta2, krem = scan_level(512, krem)
    p2 = (beta1 << 9) | beta2
    lax.fori_loop(0, 512, zero_body, 0, unroll=8)
    fill_level(4, 0x1FF, p2, 13)
    beta3, krem = scan_level(512, krem)
    p3 = (p2 << 9) | beta3
    lax.fori_loop(0, 16, zero_body, 0, unroll=8)
    fill_level(0, 0xF, p3, 4)
    beta4, krem = scan_level(16, krem)
    t_bits = (p3 << 4) | beta4

    t_splat = jnp.full((16,), t_bits, jnp.int32)

    def pass_f(i, carry):
        sgt_acc, cgt_acc = carry
        cv = loss_v[pl.ds(i * 16, 16)]
        bits = plsc.bitcast(cv, jnp.int32)
        g = bits > t_splat
        sgt_acc = sgt_acc + jnp.where(g, cv, jnp.float32(0.0))
        cgt_acc = cgt_acc + jnp.where(g, jnp.int32(1), jnp.int32(0))
        return sgt_acc, cgt_acc

    sgt_acc, cgt_acc = lax.fori_loop(0, nchunk, pass_f, (zs, zc), unroll=8)
    s_gt = jnp.sum(sgt_acc)
    cnt_gt = jnp.sum(cgt_acc)

    t_vec = plsc.bitcast(t_splat, jnp.float32)
    kr_vec = jnp.full((16,), k - cnt_gt, jnp.int32).astype(jnp.float32)
    tie = jnp.where(jnp.full((16,), k, jnp.int32) > 0,
                    t_vec * kr_vec, jnp.float32(0.0))
    out_v[...] = jnp.full((16,), pos_sum + s_gt, jnp.float32) + tie
    pltpu.sync_copy(out_v, out_hbm.at[wid])


def kernel(pred_loc, pred_bclass, true_loc_vec, true_bclass):
    del pred_loc, true_loc_vec
    b, c, n = pred_bclass.shape
    pb_t = jnp.transpose(pred_bclass, (1, 0, 2))

    rb = 8
    nb = 4096
    nt = pl.cdiv(n, nb)
    loss = pl.pallas_call(
        _ce_kernel,
        grid=(b // rb, nt),
        in_specs=[
            pl.BlockSpec((c, rb, nb), lambda i, j: (0, i, j)),
            pl.BlockSpec((rb, nb), lambda i, j: (i, j)),
        ],
        out_specs=pl.BlockSpec((rb, nb), lambda i, j: (i, j)),
        out_shape=jax.ShapeDtypeStruct((b, n), jnp.float32),
    )(pb_t, true_bclass)

    sc_select = pl.kernel(
        _sc_select_body,
        out_type=jax.ShapeDtypeStruct((b, 16), jnp.float32),
        mesh=plsc.VectorSubcoreMesh(
            core_axis_name="c", subcore_axis_name="s",
            num_cores=_SC_CORES, num_subcores=_SC_SUBCORES),
        scratch_types=[
            pltpu.VMEM((n,), jnp.float32),
            pltpu.VMEM((n,), jnp.int32),
            pltpu.VMEM((8192,), jnp.int32),
            pltpu.VMEM((16,), jnp.float32),
        ],
        compiler_params=pltpu.CompilerParams(needs_layout_passes=False),
    )
    out = sc_select(loss, true_bclass)
    return out[:, 0]


# speedup vs baseline: 1.3367x; 1.3037x over previous
import jax
import jax.numpy as jnp
from jax import lax
from jax.experimental import pallas as pl
from jax.experimental.pallas import tpu as pltpu
from jax.experimental.pallas import tpu_sc as plsc

_SC_CORES = 2
_SC_SUBCORES = 16


def _ce_kernel(logits_ref, tgt_ref, loss_ref):
    x = logits_ref[...]
    t = tgt_ref[...]
    m = jnp.max(x, axis=0, keepdims=True)
    e = jnp.exp(x - m)
    s = jnp.sum(e, axis=0, keepdims=True)
    lse = m[0] + jnp.log(s[0])
    cls = jax.lax.broadcasted_iota(jnp.int32, x.shape, 0)
    tl = jnp.sum(jnp.where(cls == t[None], x, 0.0), axis=0)
    loss_ref[...] = lse - tl


def _sc_select_body(loss_hbm, tgt_hbm, out_hbm, loss_v, tgt_v, hist_c, out_v):
    n = loss_v.shape[0]
    nchunk = n // 16
    wid = lax.axis_index("s") * _SC_CORES + lax.axis_index("c")
    pltpu.sync_copy(loss_hbm.at[wid], loss_v)
    pltpu.sync_copy(tgt_hbm.at[wid], tgt_v)
    lane = lax.iota(jnp.int32, 16)
    zc = jnp.zeros((16,), jnp.int32)
    zs = jnp.zeros((16,), jnp.float32)
    one = jnp.ones((16,), jnp.int32)

    def zero_body(d, _):
        hist_c[pl.ds(d * 16, 16)] = zc
        return 0

    lax.fori_loop(0, 512, zero_body, 0, unroll=8)

    def pass_a(i, carry):
        cnt_acc, sum_acc = carry
        lv = loss_v[pl.ds(i * 16, 16)]
        tv = tgt_v[pl.ds(i * 16, 16)]
        m = tv > 0
        cnt_acc = cnt_acc + jnp.where(m, jnp.int32(1), jnp.int32(0))
        sum_acc = sum_acc + jnp.where(m, lv, jnp.float32(0.0))
        con = jnp.where(m, jnp.float32(0.0), jnp.maximum(lv, jnp.float32(0.0)))
        loss_v[pl.ds(i * 16, 16)] = con
        bits = plsc.bitcast(con, jnp.int32)
        idx = ((bits >> 22) << 4) + lane
        plsc.addupdate_scatter(hist_c, [idx], one)
        return cnt_acc, sum_acc

    cnt_acc, sum_acc = plsc.parallel_loop(
        0, nchunk, 1, unroll=8,
        carry=(jnp.zeros((16,), jnp.int32), jnp.zeros((16,), jnp.float32)),
        )(pass_a)
    pos_num = jnp.sum(cnt_acc)
    pos_sum = jnp.sum(sum_acc)
    k = jnp.minimum(3 * pos_num, n)

    def fill_level(shift, dmask, prefix, mask_shift):
        def body(i):
            cv = loss_v[pl.ds(i * 16, 16)]
            bits = plsc.bitcast(cv, jnp.int32)
            m = (bits >> mask_shift) == prefix
            idx = (((bits >> shift) & dmask) << 4) + lane
            plsc.addupdate_scatter(hist_c, [idx], one, mask=m)

        plsc.parallel_loop(0, nchunk, 1, unroll=8)(body)

    def scan_level(nbuckets, k_rem):
        nch = nbuckets // 16

        def chunk_body(j, carry):
            found, cstar, krem_c, cum = carry
            c = nch - 1 - j
            base = c * 256

            def acc_body(l, a):
                return a + hist_c[pl.ds(base + l * 16, 16)]

            cc = lax.fori_loop(0, 16, acc_body, zc, unroll=16)
            tot_c = jnp.sum(cc)
            newcum = cum + tot_c
            cross = jnp.logical_and(jnp.logical_not(found), newcum >= k_rem)
            cstar = jnp.where(cross, c, cstar)
            krem_c = jnp.where(cross, k_rem - cum, krem_c)
            found = jnp.logical_or(found, cross)
            return found, cstar, krem_c, newcum

        init = (jnp.bool_(False), jnp.int32(0), jnp.int32(0), jnp.int32(0))
        _, cstar, krem_c, _ = lax.fori_loop(0, nch, chunk_body, init)

        def bucket_body(j, carry):
            found, dstar, krem_d, cum = carry
            d = cstar * 16 + (15 - j)
            c_d = jnp.sum(hist_c[pl.ds(d * 16, 16)])
            newcum = cum + c_d
            cross = jnp.logical_and(jnp.logical_not(found), newcum >= krem_c)
            dstar = jnp.where(cross, d, dstar)
            krem_d = jnp.where(cross, krem_c - cum, krem_d)
            found = jnp.logical_or(found, cross)
            return found, dstar, krem_d, newcum

        init = (jnp.bool_(False), jnp.int32(0), jnp.int32(0), jnp.int32(0))
        _, beta, krem, _ = lax.fori_loop(0, 16, bucket_body, init)
        return beta, krem

    beta1, krem = scan_level(512, k)
    lax.fori_loop(0, 512, zero_body, 0, unroll=8)
    fill_level(13, 0x1FF, beta1, 22)
    beta2, krem = scan_level(512, krem)
    p2 = (beta1 << 9) | beta2
    lax.fori_loop(0, 512, zero_body, 0, unroll=8)
    fill_level(4, 0x1FF, p2, 13)
    beta3, krem = scan_level(512, krem)
    p3 = (p2 << 9) | beta3
    lax.fori_loop(0, 16, zero_body, 0, unroll=8)
    fill_level(0, 0xF, p3, 4)
    beta4, krem = scan_level(16, krem)
    t_bits = (p3 << 4) | beta4

    t_splat = jnp.full((16,), t_bits, jnp.int32)

    def pass_f(i, carry):
        sgt_acc, cgt_acc = carry
        cv = loss_v[pl.ds(i * 16, 16)]
        bits = plsc.bitcast(cv, jnp.int32)
        g = bits > t_splat
        sgt_acc = sgt_acc + jnp.where(g, cv, jnp.float32(0.0))
        cgt_acc = cgt_acc + jnp.where(g, jnp.int32(1), jnp.int32(0))
        return sgt_acc, cgt_acc

    sgt_acc, cgt_acc = plsc.parallel_loop(0, nchunk, 1, unroll=8,
                                          carry=(zs, zc))(pass_f)
    s_gt = jnp.sum(sgt_acc)
    cnt_gt = jnp.sum(cgt_acc)

    t_vec = plsc.bitcast(t_splat, jnp.float32)
    kr_vec = jnp.full((16,), k - cnt_gt, jnp.int32).astype(jnp.float32)
    tie = jnp.where(jnp.full((16,), k, jnp.int32) > 0,
                    t_vec * kr_vec, jnp.float32(0.0))
    out_v[...] = jnp.full((16,), pos_sum + s_gt, jnp.float32) + tie
    pltpu.sync_copy(out_v, out_hbm.at[wid])


def kernel(pred_loc, pred_bclass, true_loc_vec, true_bclass):
    del pred_loc, true_loc_vec
    b, c, n = pred_bclass.shape
    pb_t = jnp.transpose(pred_bclass, (1, 0, 2))

    rb = 8
    nb = 4096
    nt = pl.cdiv(n, nb)
    loss = pl.pallas_call(
        _ce_kernel,
        grid=(b // rb, nt),
        in_specs=[
            pl.BlockSpec((c, rb, nb), lambda i, j: (0, i, j)),
            pl.BlockSpec((rb, nb), lambda i, j: (i, j)),
        ],
        out_specs=pl.BlockSpec((rb, nb), lambda i, j: (i, j)),
        out_shape=jax.ShapeDtypeStruct((b, n), jnp.float32),
    )(pb_t, true_bclass)

    sc_select = pl.kernel(
        _sc_select_body,
        out_type=jax.ShapeDtypeStruct((b, 16), jnp.float32),
        mesh=plsc.VectorSubcoreMesh(
            core_axis_name="c", subcore_axis_name="s",
            num_cores=_SC_CORES, num_subcores=_SC_SUBCORES),
        scratch_types=[
            pltpu.VMEM((n,), jnp.float32),
            pltpu.VMEM((n,), jnp.int32),
            pltpu.VMEM((8192,), jnp.int32),
            pltpu.VMEM((16,), jnp.float32),
        ],
        compiler_params=pltpu.CompilerParams(needs_layout_passes=False),
    )
    out = sc_select(loss, true_bclass)
    return out[:, 0]
